# baseline (speedup 1.0000x reference)
import functools

import jax
import jax.numpy as jnp
from jax import lax
from jax.experimental import pallas as pl
from jax.experimental.pallas import tpu as pltpu
from jax.experimental.pallas import tpu_sc as plsc

D = 32
NW = 32
LANES = 16


def _make_kernel(B, H, n_rows):
    hblks, hsub_n = H // 8, 8
    bblks = B // 128
    bb_per_w = bblks // NW
    dblks = D // 8
    n_steps = H
    mesh = plsc.VectorSubcoreMesh(core_axis_name="c", subcore_axis_name="s")

    @functools.partial(
        pl.kernel,
        mesh=mesh,
        out_type=jax.ShapeDtypeStruct((H, dblks, bblks, 8, 128), jnp.float32),
        scratch_types=[
            pltpu.VMEM((n_rows * D,), jnp.float32),
            pltpu.VMEM((bb_per_w, 128), jnp.int32),
            pltpu.VMEM((bb_per_w, 128), jnp.int32),
            pltpu.VMEM((dblks, bb_per_w, 8, 128), jnp.float32),
            pltpu.VMEM((dblks, bb_per_w, 8, 128), jnp.float32),
        ]
        + [pltpu.SemaphoreType.DMA] * 5,
        compiler_params=pltpu.CompilerParams(
            use_tc_tiling_on_sc=False, needs_layout_passes=False),
    )
    def k(w_hbm, t_hbm, out_hbm, table_v, idx0, idx1, rows0, rows1, *sems):
        sem_t = sems[0]
        sem_i = sems[1:3]
        sem_o = sems[3:5]
        idx_b = [idx0, idx1]
        rows_b = [rows0, rows1]
        wid = lax.axis_index("s") * 2 + lax.axis_index("c")
        bb0 = wid * bb_per_w

        def idx_desc(h, b):
            return pltpu.make_async_copy(
                w_hbm.at[h // 8, pl.ds(bb0, bb_per_w), h % 8],
                idx_b[b], sem_i[b])

        def out_descs(h, b):
            return [
                pltpu.make_async_copy(
                    rows_b[b].at[dblk],
                    out_hbm.at[h, dblk, pl.ds(bb0, bb_per_w)],
                    sem_o[b])
                for dblk in range(dblks)
            ]

        tab_desc = pltpu.make_async_copy(t_hbm, table_v, sem_t)
        tab_desc.start()
        idx_desc(0, 0).start()
        idx_desc(1, 1).start()
        tab_desc.wait()

        def compute(b):
            iv = idx_b[b]
            rv = rows_b[b]
            for bb in range(bb_per_w):
                for j in range(128 // LANES):
                    idxv = iv[bb, pl.ds(j * LANES, LANES)]
                    for d0 in range(0, D, 8):
                        gs = [
                            plsc.load_gather(table_v, [idxv + (d0 + k) * n_rows])
                            for k in range(8)
                        ]
                        for k, g in enumerate(gs):
                            d = d0 + k
                            rv[d // 8, bb, d % 8, pl.ds(j * LANES, LANES)] = g

        def step(h, b, p):
            idx_desc(h, b).wait()

            @pl.when(p > 0)
            def _drain():
                for dsc in out_descs(h - 2, b):
                    dsc.wait()

            compute(b)
            for dsc in out_descs(h, b):
                dsc.start()
            idx_desc(jnp.minimum(h + 2, n_steps - 1), b).start()

        def body(p, carry):
            step(p * 2, 0, p)
            step(p * 2 + 1, 1, p)
            return carry

        lax.fori_loop(0, n_steps // 2, body, 0)

        for dsc in out_descs(n_steps - 2, 0):
            dsc.wait()
        for dsc in out_descs(n_steps - 1, 1):
            dsc.wait()
        idx_desc(n_steps - 1, 0).wait()
        idx_desc(n_steps - 1, 1).wait()

    return k


def kernel(widths, table):
    B, H = widths.shape
    n_rows = table.shape[0]
    w4 = widths.reshape(B // 128, 128, H // 8, 8).transpose(2, 0, 3, 1)
    tt = table.T.reshape(-1)
    out5 = _make_kernel(B, H, n_rows)(w4, tt)
    return out5.transpose(2, 4, 0, 1, 3).reshape(B, H, D)


# speedup vs baseline: 2.2584x; 2.2584x over previous
import functools

import jax
import jax.numpy as jnp
from jax import lax
from jax.experimental import pallas as pl
from jax.experimental.pallas import tpu as pltpu
from jax.experimental.pallas import tpu_sc as plsc

D = 32
NW = 32
LANES = 16


def _make_kernel(B, H, n_rows):
    hblks, hsub_n = H // 8, 8
    bblks = B // 128
    bb_per_w = bblks // NW
    dblks = D // 8
    n_steps = H
    mesh = plsc.VectorSubcoreMesh(core_axis_name="c", subcore_axis_name="s")

    @functools.partial(
        pl.kernel,
        mesh=mesh,
        out_type=jax.ShapeDtypeStruct((H, dblks, bblks, 8, 128), jnp.float32),
        scratch_types=[
            pltpu.VMEM((n_rows * D,), jnp.float32),
            pltpu.VMEM((bb_per_w, 128), jnp.int32),
            pltpu.VMEM((bb_per_w, 128), jnp.int32),
            pltpu.VMEM((dblks, bb_per_w, 8, 128), jnp.float32),
            pltpu.VMEM((dblks, bb_per_w, 8, 128), jnp.float32),
        ]
        + [pltpu.SemaphoreType.DMA] * 5,
        compiler_params=pltpu.CompilerParams(
            use_tc_tiling_on_sc=False, needs_layout_passes=False),
    )
    def k(w_hbm, t_hbm, out_hbm, table_v, idx0, idx1, rows0, rows1, *sems):
        sem_t = sems[0]
        sem_i = sems[1:3]
        sem_o = sems[3:5]
        idx_b = [idx0, idx1]
        rows_b = [rows0, rows1]
        wid = lax.axis_index("s") * 2 + lax.axis_index("c")
        bb0 = wid * bb_per_w

        def idx_desc(h, b):
            return pltpu.make_async_copy(
                w_hbm.at[h // 8, pl.ds(bb0, bb_per_w), h % 8],
                idx_b[b], sem_i[b])

        def out_descs(h, b):
            return [
                pltpu.make_async_copy(
                    rows_b[b].at[dblk],
                    out_hbm.at[h, dblk, pl.ds(bb0, bb_per_w)],
                    sem_o[b])
                for dblk in range(dblks)
            ]

        tab_desc = pltpu.make_async_copy(t_hbm, table_v, sem_t)
        tab_desc.start()
        idx_desc(0, 0).start()
        idx_desc(1, 1).start()
        tab_desc.wait()

        def compute(b):
            iv = idx_b[b]
            rv = rows_b[b]
            for bb in range(bb_per_w):
                def jbody(j, carry, bb=bb):
                    idxv = iv[bb, pl.ds(j * LANES, LANES)]
                    for d0 in range(0, D, 16):
                        gs = [
                            plsc.load_gather(table_v, [idxv + (d0 + k) * n_rows])
                            for k in range(16)
                        ]
                        for k, g in enumerate(gs):
                            d = d0 + k
                            rv[d // 8, bb, d % 8, pl.ds(j * LANES, LANES)] = g
                    return carry

                lax.fori_loop(0, 128 // LANES, jbody, 0)

        def step(h, b, p):
            idx_desc(h, b).wait()

            @pl.when(p > 0)
            def _drain():
                for dsc in out_descs(h - 2, b):
                    dsc.wait()

            compute(b)
            for dsc in out_descs(h, b):
                dsc.start()
            idx_desc(jnp.minimum(h + 2, n_steps - 1), b).start()

        def body(p, carry):
            step(p * 2, 0, p)
            step(p * 2 + 1, 1, p)
            return carry

        lax.fori_loop(0, n_steps // 2, body, 0)

        for dsc in out_descs(n_steps - 2, 0):
            dsc.wait()
        for dsc in out_descs(n_steps - 1, 1):
            dsc.wait()
        idx_desc(n_steps - 1, 0).wait()
        idx_desc(n_steps - 1, 1).wait()

    return k


def kernel(widths, table):
    B, H = widths.shape
    n_rows = table.shape[0]
    w4 = widths.reshape(B // 128, 128, H // 8, 8).transpose(2, 0, 3, 1)
    tt = table.T.reshape(-1)
    out5 = _make_kernel(B, H, n_rows)(w4, tt)
    return out5.transpose(2, 4, 0, 1, 3).reshape(B, H, D)


# speedup vs baseline: 2.2807x; 1.0099x over previous
import functools

import jax
import jax.numpy as jnp
from jax import lax
from jax.experimental import pallas as pl
from jax.experimental.pallas import tpu as pltpu
from jax.experimental.pallas import tpu_sc as plsc

D = 32
NW = 32
LANES = 16


def _make_kernel(B, H, n_rows):
    hblks, hsub_n = H // 8, 8
    bblks = B // 128
    bb_per_w = bblks // NW
    dblks = D // 8
    n_steps = H
    mesh = plsc.VectorSubcoreMesh(core_axis_name="c", subcore_axis_name="s")

    @functools.partial(
        pl.kernel,
        mesh=mesh,
        out_type=jax.ShapeDtypeStruct((H, dblks, bblks, 8, 128), jnp.float32),
        scratch_types=[
            pltpu.VMEM((n_rows * D,), jnp.float32),
            pltpu.VMEM((bb_per_w, 128), jnp.int32),
            pltpu.VMEM((bb_per_w, 128), jnp.int32),
            pltpu.VMEM((dblks, bb_per_w, 8, 128), jnp.float32),
            pltpu.VMEM((dblks, bb_per_w, 8, 128), jnp.float32),
        ]
        + [pltpu.SemaphoreType.DMA] * 5,
        compiler_params=pltpu.CompilerParams(
            use_tc_tiling_on_sc=False, needs_layout_passes=False),
    )
    def k(w_hbm, t_hbm, out_hbm, table_v, idx0, idx1, rows0, rows1, *sems):
        sem_t = sems[0]
        sem_i = sems[1:3]
        sem_o = sems[3:5]
        idx_b = [idx0, idx1]
        rows_b = [rows0, rows1]
        wid = lax.axis_index("s") * 2 + lax.axis_index("c")
        bb0 = wid * bb_per_w

        def idx_desc(h, b):
            return pltpu.make_async_copy(
                w_hbm.at[h // 8, pl.ds(bb0, bb_per_w), h % 8],
                idx_b[b], sem_i[b])

        def out_descs(h, b):
            return [
                pltpu.make_async_copy(
                    rows_b[b].at[dblk],
                    out_hbm.at[h, dblk, pl.ds(bb0, bb_per_w)],
                    sem_o[b])
                for dblk in range(dblks)
            ]

        tab_desc = pltpu.make_async_copy(t_hbm, table_v, sem_t)
        tab_desc.start()
        idx_desc(0, 0).start()
        idx_desc(1, 1).start()
        tab_desc.wait()

        def compute(b):
            iv = idx_b[b]
            rv = rows_b[b]
            for bb in range(bb_per_w):
                def jbody(j, carry, bb=bb):
                    idxv = iv[bb, pl.ds(j * LANES, LANES)]
                    for d0 in range(0, D, 16):
                        gs = [
                            plsc.load_gather(table_v, [idxv + (d0 + k) * n_rows])
                            for k in range(16)
                        ]
                        for k, g in enumerate(gs):
                            d = d0 + k
                            rv[d // 8, bb, d % 8, pl.ds(j * LANES, LANES)] = g
                    return carry

                lax.fori_loop(0, 128 // LANES, jbody, 0, unroll=2)

        def step(h, b, p):
            idx_desc(h, b).wait()

            @pl.when(p > 0)
            def _drain():
                for dsc in out_descs(h - 2, b):
                    dsc.wait()

            compute(b)
            for dsc in out_descs(h, b):
                dsc.start()
            idx_desc(jnp.minimum(h + 2, n_steps - 1), b).start()

        def body(p, carry):
            step(p * 2, 0, p)
            step(p * 2 + 1, 1, p)
            return carry

        lax.fori_loop(0, n_steps // 2, body, 0)

        for dsc in out_descs(n_steps - 2, 0):
            dsc.wait()
        for dsc in out_descs(n_steps - 1, 1):
            dsc.wait()
        idx_desc(n_steps - 1, 0).wait()
        idx_desc(n_steps - 1, 1).wait()

    return k


def kernel(widths, table):
    B, H = widths.shape
    n_rows = table.shape[0]
    w4 = widths.reshape(B // 128, 128, H // 8, 8).transpose(2, 0, 3, 1)
    tt = table.T.reshape(-1)
    out5 = _make_kernel(B, H, n_rows)(w4, tt)
    return out5.transpose(2, 4, 0, 1, 3).reshape(B, H, D)
